# baseline (speedup 1.0000x reference)
import functools

import jax
import jax.numpy as jnp
from jax import lax
from jax.experimental import pallas as pl
from jax.experimental.pallas import tpu as pltpu
from jax.experimental.pallas import tpu_sc as plsc

_VOCAB = 100000
_NC, _NS = 2, 16
_NW = _NC * _NS
_D = 64
_PPW = _D // _NW


def _sc_gather_t(idx1d, table1d, b, s):
  mesh = plsc.VectorSubcoreMesh(core_axis_name="c", subcore_axis_name="s")
  n_grp = s // 2

  @functools.partial(
      pl.kernel,
      out_type=jax.ShapeDtypeStruct((s * _D * b,), jnp.float32),
      mesh=mesh,
      scratch_types=[
          pltpu.VMEM((_VOCAB,), jnp.float32),
          pltpu.VMEM((2, b), jnp.int32),
          pltpu.VMEM((2, b), jnp.float32),
          pltpu.SemaphoreType.DMA,
          pltpu.SemaphoreType.DMA,
          pltpu.SemaphoreType.DMA,
      ],
      compiler_params=pltpu.CompilerParams(
          use_tc_tiling_on_sc=False, needs_layout_passes=False),
  )
  def k(idx_hbm, table_hbm, out_hbm, plane_v, idx_v, ob_v, isem, osem0, osem1):
    wid = lax.axis_index("s") * _NC + lax.axis_index("c")
    osems = (osem0, osem1)

    def idx_start(si, t):
      return pltpu.async_copy(idx_hbm.at[pl.ds(si * b, b)], idx_v.at[t], isem)

    def gather(t):
      @plsc.parallel_loop(0, b, 16, unroll=16)
      def _(i):
        iv = idx_v[t, pl.ds(i, 16)]
        ob_v[t, pl.ds(i, 16)] = plsc.load_gather(plane_v, [iv])

    def out_start(si, p, t):
      return pltpu.async_copy(
          ob_v.at[t], out_hbm.at[pl.ds((si * _D + p) * b, b)], osems[t])

    for pi in range(_PPW):
      p = wid * _PPW + pi
      pltpu.sync_copy(table_hbm.at[pl.ds(p * _VOCAB, _VOCAB)], plane_v)

      idx_start(0, 0).wait()
      idx_start(1, 1)
      gather(0)
      od0 = out_start(0, p, 0)
      pltpu.make_async_copy(idx_hbm.at[pl.ds(b, b)], idx_v.at[1], isem).wait()
      idx_start(2, 0)
      gather(1)
      od1 = out_start(1, p, 1)

      def grp(g, carry):
        si = 2 * g
        pltpu.make_async_copy(
            idx_hbm.at[pl.ds(si * b, b)], idx_v.at[0], isem).wait()
        idx_start(si + 1, 1)
        pltpu.make_async_copy(
            ob_v.at[0], out_hbm.at[pl.ds(b, b)], osem0).wait()
        gather(0)
        out_start(si, p, 0)
        pltpu.make_async_copy(
            idx_hbm.at[pl.ds((si + 1) * b, b)], idx_v.at[1], isem).wait()
        idx_start(si + 2, 0)
        pltpu.make_async_copy(
            ob_v.at[1], out_hbm.at[pl.ds(b, b)], osem1).wait()
        gather(1)
        out_start(si + 1, p, 1)
        return carry

      lax.fori_loop(1, n_grp - 1, grp, 0)

      si = s - 2
      pltpu.make_async_copy(
          idx_hbm.at[pl.ds(si * b, b)], idx_v.at[0], isem).wait()
      idx_start(si + 1, 1)
      pltpu.make_async_copy(
          ob_v.at[0], out_hbm.at[pl.ds(b, b)], osem0).wait()
      gather(0)
      odl0 = out_start(si, p, 0)
      pltpu.make_async_copy(
          idx_hbm.at[pl.ds((si + 1) * b, b)], idx_v.at[1], isem).wait()
      pltpu.make_async_copy(
          ob_v.at[1], out_hbm.at[pl.ds(b, b)], osem1).wait()
      gather(1)
      odl1 = out_start(si + 1, p, 1)
      odl0.wait()
      odl1.wait()

  return k(idx1d, table1d)


def kernel(inputs, kernel):
  b, s = inputs.shape
  idx1d = inputs.T.reshape(s * b).astype(jnp.int32)
  table1d = kernel.transpose(1, 2, 0).reshape(_D * _VOCAB)
  out1d = _sc_gather_t(idx1d, table1d, b, s)
  return out1d.reshape(s, 8, 8, b).transpose(3, 0, 1, 2)


# speedup vs baseline: 1.1691x; 1.1691x over previous
import functools

import jax
import jax.numpy as jnp
from jax import lax
from jax.experimental import pallas as pl
from jax.experimental.pallas import tpu as pltpu
from jax.experimental.pallas import tpu_sc as plsc

_VOCAB = 100000
_NC, _NS = 2, 16
_NW = _NC * _NS
_D = 64
_PPW = _D // _NW
_NBUF = 3


def _sc_gather_t(idx1d, table1d, b, s):
  mesh = plsc.VectorSubcoreMesh(core_axis_name="c", subcore_axis_name="s")
  assert s % _NBUF == 2

  @functools.partial(
      pl.kernel,
      out_type=jax.ShapeDtypeStruct((s * _D * b,), jnp.float32),
      mesh=mesh,
      scratch_types=[
          pltpu.VMEM((_VOCAB,), jnp.float32),
          pltpu.VMEM((_NBUF, b), jnp.int32),
          pltpu.VMEM((_NBUF, b), jnp.float32),
          pltpu.SemaphoreType.DMA,
          pltpu.SemaphoreType.DMA,
          pltpu.SemaphoreType.DMA,
          pltpu.SemaphoreType.DMA,
      ],
      compiler_params=pltpu.CompilerParams(
          use_tc_tiling_on_sc=False, needs_layout_passes=False),
  )
  def k(idx_hbm, table_hbm, out_hbm, plane_v, idx_v, ob_v, isem, os0, os1, os2):
    wid = lax.axis_index("s") * _NC + lax.axis_index("c")
    osems = (os0, os1, os2)

    def idx_start(si, t):
      return pltpu.async_copy(idx_hbm.at[pl.ds(si * b, b)], idx_v.at[t], isem)

    def idx_wait(t):
      pltpu.make_async_copy(idx_hbm.at[pl.ds(0, b)], idx_v.at[t], isem).wait()

    def gather(t):
      @plsc.parallel_loop(0, b, 16, unroll=8)
      def _(i):
        iv = idx_v[t, pl.ds(i, 16)]
        ob_v[t, pl.ds(i, 16)] = plsc.load_gather(plane_v, [iv])

    def out_start(si, p, t):
      return pltpu.async_copy(
          ob_v.at[t], out_hbm.at[pl.ds((si * _D + p) * b, b)], osems[t])

    def out_wait(t):
      pltpu.make_async_copy(ob_v.at[t], out_hbm.at[pl.ds(0, b)], osems[t]).wait()

    def step(si, p, t, first_round, prefetch):
      idx_wait(t)
      if not first_round:
        out_wait(t)
      gather(t)
      out_start(si, p, t)
      if prefetch:
        idx_start(si + _NBUF, t)

    for pi in range(_PPW):
      p = wid * _PPW + pi
      pltpu.sync_copy(table_hbm.at[pl.ds(p * _VOCAB, _VOCAB)], plane_v)

      for t in range(_NBUF):
        idx_start(t, t)
      for t in range(_NBUF):
        step(t, p, t, first_round=True, prefetch=True)

      def grp(g, carry):
        si = _NBUF * g
        for t in range(_NBUF):
          step(si + t, p, t, first_round=False, prefetch=True)
        return carry

      n_grp = (s - 2) // _NBUF
      lax.fori_loop(1, n_grp - 1, grp, 0)

      si = (n_grp - 1) * _NBUF
      step(si, p, 0, first_round=False, prefetch=True)
      step(si + 1, p, 1, first_round=False, prefetch=True)
      step(si + 2, p, 2, first_round=False, prefetch=False)

      si = s - 2
      step(si, p, 0, first_round=False, prefetch=False)
      step(si + 1, p, 1, first_round=False, prefetch=False)
      out_wait(2)
      out_wait(0)
      out_wait(1)

  return k(idx1d, table1d)


def kernel(inputs, kernel):
  b, s = inputs.shape
  idx1d = inputs.T.reshape(s * b).astype(jnp.int32)
  table1d = kernel.transpose(1, 2, 0).reshape(_D * _VOCAB)
  out1d = _sc_gather_t(idx1d, table1d, b, s)
  return out1d.reshape(s, 8, 8, b).transpose(3, 0, 1, 2)
